# baseline (speedup 1.0000x reference)
import jax
import jax.numpy as jnp
from jax import lax
from jax.experimental import pallas as pl
from jax.experimental.pallas import tpu as pltpu
from jax.experimental.pallas import tpu_sc as plsc

D = 64
NC, NS, L = 2, 16, 16
NW = NC * NS

B = 4096
C = 200
RB = B // 128
N_ITEMS = C * RB
ITEMS_PER_W = N_ITEMS // NW


def _transpose_block(g, idx_v, t, rv):
    def body(h, c2):
        hb = h // 8
        hm = h % 8
        hv = lax.broadcast(h, (L,))
        for j in range(8):
            v = plsc.load_gather(g, [rv[j], hv])
            t[hb, hm, pl.ds(j * L, L)] = v
        return c2

    lax.fori_loop(0, D, body, 0, unroll=False)

    def mred(j, acc):
        return acc | (idx_v[pl.ds(j * L, L)] == 0)

    mv = lax.fori_loop(1, 8, mred, idx_v[pl.ds(0, L)] == 0)
    nz = plsc.all_reduce_population_count(mv)[0]

    @pl.when(nz > 0)
    def _fixup():
        zeros = jnp.zeros((L,), jnp.float32)

        def fix_group(j, c3):
            m = idx_v[pl.ds(j * L, L)] == 0
            rv = j * L + lax.iota(jnp.int32, L)

            def fh(h, c4):
                rows = lax.broadcast(h // 8, (L,))
                mids = lax.broadcast(h % 8, (L,))
                plsc.store_scatter(t, [rows, mids, rv], zeros, mask=m)
                return c4

            lax.fori_loop(0, D, fh, 0)
            return c3

        lax.fori_loop(0, 8, fix_group, 0)


def _emb_body(table_hbm, idx_hbm, out_hbm,
              idx0, idx1, g0, g1, t0, t1,
              isem0, isem1, gsem0, gsem1, wsem0, wsem1):
    wid = lax.axis_index("s") * NC + lax.axis_index("c")
    m0 = wid * ITEMS_PER_W

    idx_bufs = (idx0, idx1)
    g_bufs = (g0, g1)
    t_bufs = (t0, t1)
    isems = (isem0, isem1)
    gsems = (gsem0, gsem1)
    wsems = (wsem0, wsem1)

    lane = lax.iota(jnp.int32, L)
    rv = tuple(j * L + lane for j in range(8))

    def idx_src(j):
        m = m0 + j
        c = m // RB
        rb = m % RB
        return idx_hbm.at[pl.ds(c * B + rb * 128, 128)]

    def out_dst(j):
        m = m0 + j
        c = m // RB
        rb = m % RB
        return out_hbm.at[c, :, rb]

    pltpu.async_copy(idx_src(0), idx0, isem0)
    pltpu.async_copy(idx_src(1), idx1, isem1)
    pltpu.make_async_copy(idx_src(0), idx0, isem0).wait()
    pltpu.async_copy(table_hbm.at[idx0], g0, gsem0)

    def phase(j, p):
        q = 1 - p

        pltpu.make_async_copy(table_hbm.at[idx_bufs[p]], g_bufs[p],
                              gsems[p]).wait()

        @pl.when(j + 2 < ITEMS_PER_W)
        def _pref():
            pltpu.async_copy(idx_src(j + 2), idx_bufs[p], isems[p])

        @pl.when(j + 1 < ITEMS_PER_W)
        def _next():
            pltpu.make_async_copy(idx_src(j + 1), idx_bufs[q],
                                  isems[q]).wait()
            pltpu.async_copy(table_hbm.at[idx_bufs[q]], g_bufs[q],
                             gsems[q])

        @pl.when(j >= 2)
        def _drain():
            pltpu.make_async_copy(t_bufs[p], out_dst(j - 2),
                                  wsems[p]).wait()

        _transpose_block(g_bufs[p], idx_bufs[p], t_bufs[p], rv)

        pltpu.async_copy(t_bufs[p], out_dst(j), wsems[p])

    def body(i, carry):
        phase(i * 2, 0)
        phase(i * 2 + 1, 1)
        return carry

    lax.fori_loop(0, ITEMS_PER_W // 2, body, 0)

    pltpu.make_async_copy(t0, out_dst(ITEMS_PER_W - 2), wsems[0]).wait()
    pltpu.make_async_copy(t1, out_dst(ITEMS_PER_W - 1), wsems[1]).wait()


@jax.jit
def _emb(idx_t, table):
    mesh = plsc.VectorSubcoreMesh(core_axis_name="c", subcore_axis_name="s")
    f = pl.kernel(
        _emb_body,
        out_type=jax.ShapeDtypeStruct((C, 8, RB, 8, 128), jnp.float32),
        mesh=mesh,
        compiler_params=pltpu.CompilerParams(needs_layout_passes=False,
                                             use_tc_tiling_on_sc=False),
        scratch_types=[
            pltpu.VMEM((128,), jnp.int32),
            pltpu.VMEM((128,), jnp.int32),
            pltpu.VMEM((128, D), jnp.float32),
            pltpu.VMEM((128, D), jnp.float32),
            pltpu.VMEM((8, 8, 128), jnp.float32),
            pltpu.VMEM((8, 8, 128), jnp.float32),
            pltpu.SemaphoreType.DMA,
            pltpu.SemaphoreType.DMA,
            pltpu.SemaphoreType.DMA,
            pltpu.SemaphoreType.DMA,
            pltpu.SemaphoreType.DMA,
            pltpu.SemaphoreType.DMA,
        ],
    )
    return f(table, idx_t)


def kernel(inputs, shared_weights):
    idx_t = inputs.T.reshape(-1).astype(jnp.int32)
    out5 = _emb(idx_t, shared_weights)
    return out5.transpose(2, 4, 0, 1, 3).reshape(B, C, D)


# speedup vs baseline: 1.4710x; 1.4710x over previous
import jax
import jax.numpy as jnp
from jax import lax
from jax.experimental import pallas as pl
from jax.experimental.pallas import tpu as pltpu
from jax.experimental.pallas import tpu_sc as plsc

D = 64
NC, NS, L = 2, 16, 16
NW = NC * NS

B_TOTAL = 4096 * 200
B_PER_W = B_TOTAL // NW
CHUNK = 512
N_CHUNKS = B_PER_W // CHUNK
GRP = CHUNK // L


def _scan_and_fixup(idx_v, rows_v):
    def mred(j, acc):
        return acc | (idx_v[pl.ds(j * L, L)] == 0)

    mv = lax.fori_loop(1, GRP, mred, idx_v[pl.ds(0, L)] == 0)
    nz = plsc.all_reduce_population_count(mv)[0]

    @pl.when(nz > 0)
    def _fixup():
        zeros = jnp.zeros((L,), jnp.float32)

        def fix_group(j, c2):
            v = idx_v[pl.ds(j * L, L)]
            m = v == 0
            rowids = j * L + lax.iota(jnp.int32, L)

            def fk(k, c3):
                colids = lax.broadcast(k, (L,))
                plsc.store_scatter(rows_v, [rowids, colids], zeros, mask=m)
                return c3

            lax.fori_loop(0, D, fk, 0)
            return c2

        lax.fori_loop(0, GRP, fix_group, 0)


def _emb_body(table_hbm, idx_hbm, out_hbm,
              idx0, idx1, rows0, rows1,
              isem0, isem1, gsem0, gsem1, wsem0, wsem1):
    wid = lax.axis_index("s") * NC + lax.axis_index("c")
    base0 = wid * B_PER_W

    idx_bufs = (idx0, idx1)
    rows_bufs = (rows0, rows1)
    isems = (isem0, isem1)
    gsems = (gsem0, gsem1)
    wsems = (wsem0, wsem1)

    def idx_src(g):
        return idx_hbm.at[pl.ds(base0 + g * CHUNK, CHUNK)]

    def out_dst(g):
        return out_hbm.at[pl.ds(base0 + g * CHUNK, CHUNK)]

    pltpu.async_copy(idx_src(0), idx0, isem0)
    pltpu.async_copy(idx_src(1), idx1, isem1)
    pltpu.make_async_copy(idx_src(0), idx0, isem0).wait()
    pltpu.async_copy(table_hbm.at[idx0], rows0, gsem0)

    def phase(g, p):
        q = 1 - p
        idx_p, rows_p = idx_bufs[p], rows_bufs[p]
        idx_q, rows_q = idx_bufs[q], rows_bufs[q]

        pltpu.make_async_copy(table_hbm.at[idx_p], rows_p, gsems[p]).wait()

        _scan_and_fixup(idx_p, rows_p)

        @pl.when(g + 2 < N_CHUNKS)
        def _pref():
            pltpu.async_copy(idx_src(g + 2), idx_p, isems[p])

        @pl.when(g + 1 < N_CHUNKS)
        def _next():
            pltpu.make_async_copy(idx_src(g + 1), idx_q, isems[q]).wait()

            @pl.when(g >= 1)
            def _drain():
                pltpu.make_async_copy(rows_q, out_dst(g - 1),
                                      wsems[q]).wait()

            pltpu.async_copy(table_hbm.at[idx_q], rows_q, gsems[q])

        pltpu.async_copy(rows_p, out_dst(g), wsems[p])

    def body(i, carry):
        g = i * 2
        phase(g, 0)
        phase(g + 1, 1)
        return carry

    lax.fori_loop(0, N_CHUNKS // 2, body, 0)

    pltpu.make_async_copy(rows0, out_dst(N_CHUNKS - 2), wsems[0]).wait()
    pltpu.make_async_copy(rows1, out_dst(N_CHUNKS - 1), wsems[1]).wait()


def _emb(idx_flat, table):
    mesh = plsc.VectorSubcoreMesh(core_axis_name="c", subcore_axis_name="s")
    f = pl.kernel(
        _emb_body,
        out_type=jax.ShapeDtypeStruct((B_TOTAL, D), jnp.float32),
        mesh=mesh,
        compiler_params=pltpu.CompilerParams(needs_layout_passes=False,
                                             use_tc_tiling_on_sc=False),
        scratch_types=[
            pltpu.VMEM((CHUNK,), jnp.int32),
            pltpu.VMEM((CHUNK,), jnp.int32),
            pltpu.VMEM((CHUNK, D), jnp.float32),
            pltpu.VMEM((CHUNK, D), jnp.float32),
            pltpu.SemaphoreType.DMA,
            pltpu.SemaphoreType.DMA,
            pltpu.SemaphoreType.DMA,
            pltpu.SemaphoreType.DMA,
            pltpu.SemaphoreType.DMA,
            pltpu.SemaphoreType.DMA,
        ],
    )
    return f(table, idx_flat)


def kernel(inputs, shared_weights):
    idx = inputs.reshape(-1).astype(jnp.int32)
    out = _emb(idx, shared_weights)
    return out.reshape(inputs.shape + (D,))
